# baseline (speedup 1.0000x reference)
import functools

import jax
import jax.numpy as jnp
from jax import lax
from jax.experimental import pallas as pl
from jax.experimental.pallas import tpu as pltpu
from jax.experimental.pallas import tpu_sc as plsc

VOCAB = 1000000
S = 200
D = 64
B = 4096
N = B * S
NC = 2
NS = 16
NW = NC * NS
RPW = N // NW
CH = 400
G = 4
GSZ = CH // G
NCH = RPW // CH
LB = D // 16

_mesh = plsc.VectorSubcoreMesh(core_axis_name="c", subcore_axis_name="s")


@functools.partial(
    pl.kernel,
    mesh=_mesh,
    compiler_params=pltpu.CompilerParams(use_tc_tiling_on_sc=False),
    out_type=jax.ShapeDtypeStruct((N, D), jnp.float32),
    scratch_types=[
        pltpu.VMEM((G, GSZ), jnp.int32),
        pltpu.VMEM((CH, D), jnp.float32),
        pltpu.VMEM((S, D), jnp.float32),
        pltpu.SemaphoreType.DMA,
    ],
)
def _embed(x_hbm, tok_hbm, pos_hbm, out_hbm, idx_v, rows_v, pos_v, sem):
    wid = lax.axis_index("s") * NC + lax.axis_index("c")
    pltpu.sync_copy(pos_hbm, pos_v)

    def chunk_body(c, carry):
        gc = wid * NCH + c
        base = gc * CH
        pltpu.sync_copy(x_hbm.at[gc], idx_v)
        cps = [
            pltpu.async_copy(
                tok_hbm.at[idx_v.at[g]],
                rows_v.at[pl.ds(g * GSZ, GSZ)],
                sem,
            )
            for g in range(G)
        ]
        for cp in cps:
            cp.wait()

        def pos_add(s, inner):
            for lb in range(LB):
                sl = pl.ds(lb * 16, 16)
                p = pos_v[s, sl]
                for j in range(CH // S):
                    r = j * S + s
                    rows_v[r, sl] = rows_v[r, sl] + p
            return inner

        lax.fori_loop(0, S, pos_add, 0)
        pltpu.sync_copy(rows_v, out_hbm.at[pl.ds(base, CH)])
        return carry

    lax.fori_loop(0, NCH, chunk_body, 0)


def kernel(x, token_table, position_table):
    x3 = x.astype(jnp.int32).reshape(N // CH, G, GSZ)
    out = _embed(x3, token_table, position_table)
    return out.reshape(B, S, D)


# speedup vs baseline: 1.1180x; 1.1180x over previous
import functools

import jax
import jax.numpy as jnp
from jax import lax
from jax.experimental import pallas as pl
from jax.experimental.pallas import tpu as pltpu
from jax.experimental.pallas import tpu_sc as plsc

VOCAB = 1000000
S = 200
D = 64
B = 4096
N = B * S
NC = 2
NS = 16
NW = NC * NS
RPW = N // NW
CH = 400
G = 4
GSZ = CH // G
NCH = RPW // CH
LB = D // 16

_mesh = plsc.VectorSubcoreMesh(core_axis_name="c", subcore_axis_name="s")


@functools.partial(
    pl.kernel,
    mesh=_mesh,
    compiler_params=pltpu.CompilerParams(use_tc_tiling_on_sc=False),
    out_type=jax.ShapeDtypeStruct((N, D), jnp.float32),
    scratch_types=[
        pltpu.VMEM((NCH, G, GSZ), jnp.int32),
        pltpu.VMEM((2, CH, D), jnp.float32),
        pltpu.VMEM((S, D), jnp.float32),
        pltpu.SemaphoreType.DMA,
        pltpu.SemaphoreType.DMA,
        pltpu.SemaphoreType.DMA,
        pltpu.SemaphoreType.DMA,
        pltpu.SemaphoreType.DMA,
    ],
)
def _embed(x_hbm, tok_hbm, pos_hbm, out_hbm, idx_all, rows_v, pos_v,
           ssem, gsem0, gsem1, osem0, osem1):
    wid = lax.axis_index("s") * NC + lax.axis_index("c")
    base0 = wid * RPW

    cp_pos = pltpu.async_copy(pos_hbm, pos_v, ssem)
    cp_idx = pltpu.async_copy(x_hbm.at[wid], idx_all, ssem)
    cp_pos.wait()
    cp_idx.wait()

    gsems = (gsem0, gsem1)
    osems = (osem0, osem1)

    def fire_gathers(c, b, gsem):
        for g in range(G):
            pltpu.async_copy(
                tok_hbm.at[idx_all.at[c, g]],
                rows_v.at[b, pl.ds(g * GSZ, GSZ)],
                gsem,
            )

    def wait_bytes_chunk(src_like, dst_like, sem):
        pltpu.make_async_copy(src_like, dst_like, sem).wait()

    def pos_add(b):
        def body(s, inner):
            for lb in range(LB):
                sl = pl.ds(lb * 16, 16)
                p = pos_v[s, sl]
                for j in range(CH // S):
                    r = j * S + s
                    rows_v[b, r, sl] = rows_v[b, r, sl] + p
            return inner

        lax.fori_loop(0, S, body, 0)

    fire_gathers(0, 0, gsem0)

    def outer(i, carry):
        for b in range(2):
            c = 2 * i + b
            wait_bytes_chunk(out_hbm.at[pl.ds(0, CH)], rows_v.at[b], gsems[b])

            if b == 0:
                @pl.when(i >= 1)
                def _():
                    wait_bytes_chunk(rows_v.at[1], out_hbm.at[pl.ds(0, CH)],
                                     osems[1])
                fire_gathers(c + 1, 1, gsems[1])
            else:
                wait_bytes_chunk(rows_v.at[0], out_hbm.at[pl.ds(0, CH)],
                                 osems[0])

                @pl.when(i < NCH // 2 - 1)
                def _():
                    fire_gathers(c + 1, 0, gsems[0])

            pos_add(b)
            pltpu.async_copy(
                rows_v.at[b],
                out_hbm.at[pl.ds(base0 + c * CH, CH)],
                osems[b],
            )
        return carry

    lax.fori_loop(0, NCH // 2, outer, 0)
    wait_bytes_chunk(rows_v.at[1], out_hbm.at[pl.ds(0, CH)], osems[1])


def kernel(x, token_table, position_table):
    x4 = x.astype(jnp.int32).reshape(NW, NCH, G, GSZ)
    out = _embed(x4, token_table, position_table)
    return out.reshape(B, S, D)


# speedup vs baseline: 1.1189x; 1.0007x over previous
import functools

import jax
import jax.numpy as jnp
from jax import lax
from jax.experimental import pallas as pl
from jax.experimental.pallas import tpu as pltpu
from jax.experimental.pallas import tpu_sc as plsc

VOCAB = 1000000
S = 200
D = 64
B = 4096
NC = 2
NS = 16
NW = NC * NS
BPW = B // NW
CB = 2
NCH = BPW // CB
GS = (104, 96)
LB = D // 16

_mesh = plsc.VectorSubcoreMesh(core_axis_name="c", subcore_axis_name="s")


@functools.partial(
    pl.kernel,
    mesh=_mesh,
    compiler_params=pltpu.CompilerParams(use_tc_tiling_on_sc=False),
    out_type=jax.ShapeDtypeStruct((B, S, D), jnp.float32),
    scratch_types=[
        pltpu.VMEM((BPW, S), jnp.int32),
        pltpu.VMEM((2, CB, S, D), jnp.float32),
        pltpu.VMEM((S, D), jnp.float32),
        pltpu.SemaphoreType.DMA,
        pltpu.SemaphoreType.DMA,
        pltpu.SemaphoreType.DMA,
        pltpu.SemaphoreType.DMA,
        pltpu.SemaphoreType.DMA,
    ],
)
def _embed(x_hbm, tok_hbm, pos_hbm, out_hbm, idx_all, rows_v, pos_v,
           ssem, gsem0, gsem1, osem0, osem1):
    wid = lax.axis_index("s") * NC + lax.axis_index("c")
    b0 = wid * BPW

    cp_pos = pltpu.async_copy(pos_hbm, pos_v, ssem)
    cp_idx = pltpu.async_copy(x_hbm.at[pl.ds(b0, BPW)], idx_all, ssem)
    cp_pos.wait()
    cp_idx.wait()

    gsems = (gsem0, gsem1)
    osems = (osem0, osem1)

    def fire_gathers(c, bufi, gsem):
        for j in range(CB):
            off = 0
            for gsz in GS:
                pltpu.async_copy(
                    tok_hbm.at[idx_all.at[c * CB + j, pl.ds(off, gsz)]],
                    rows_v.at[bufi, j, pl.ds(off, gsz)],
                    gsem,
                )
                off += gsz

    def wait_bytes_chunk(src_like, dst_like, sem):
        pltpu.make_async_copy(src_like, dst_like, sem).wait()

    def pos_add(bufi):
        def body(s, inner):
            for lb in range(LB):
                sl = pl.ds(lb * 16, 16)
                p = pos_v[s, sl]
                for j in range(CB):
                    rows_v[bufi, j, s, sl] = rows_v[bufi, j, s, sl] + p
            return inner

        lax.fori_loop(0, S, body, 0)

    fire_gathers(0, 0, gsem0)

    def outer(i, carry):
        for bufi in range(2):
            c = 2 * i + bufi
            wait_bytes_chunk(out_hbm.at[pl.ds(0, CB)], rows_v.at[bufi],
                             gsems[bufi])

            if bufi == 0:
                @pl.when(i >= 1)
                def _():
                    wait_bytes_chunk(rows_v.at[1], out_hbm.at[pl.ds(0, CB)],
                                     osems[1])
                fire_gathers(c + 1, 1, gsems[1])
            else:
                wait_bytes_chunk(rows_v.at[0], out_hbm.at[pl.ds(0, CB)],
                                 osems[0])

                @pl.when(i < NCH // 2 - 1)
                def _():
                    fire_gathers(c + 1, 0, gsems[0])

            pos_add(bufi)
            pltpu.async_copy(
                rows_v.at[bufi],
                out_hbm.at[pl.ds(b0 + c * CB, CB)],
                osems[bufi],
            )
        return carry

    lax.fori_loop(0, NCH // 2, outer, 0)
    wait_bytes_chunk(rows_v.at[1], out_hbm.at[pl.ds(0, CB)], osems[1])


def kernel(x, token_table, position_table):
    return _embed(x.astype(jnp.int32), token_table, position_table)
